# baseline (speedup 1.0000x reference)
import functools

import jax
import jax.numpy as jnp
import numpy as np
from jax import lax
from jax.experimental import pallas as pl
from jax.experimental.pallas import tpu as pltpu
from jax.experimental.pallas import tpu_sc as plsc

_T, _B, _D = 8192, 4, 1024
_DF = _B * _D
_N1 = int(_T * 0.7)
_N2 = _T - _N1
_C = 8
_NC, _NS = 2, 16
_NW = _NC * _NS


def _ceil_div(a, b):
    return -(-a // b)


_PERM = np.asarray(jax.random.permutation(jax.random.key(42), _T))
_IDX1 = np.sort(_PERM[:_N1]).astype(np.int32)
_IDX2 = np.sort(_PERM[_N1:]).astype(np.int32)

_NB1, _NB2 = 3, 2


def _round_to(a, m):
    return _ceil_div(a, m) * m


_WPW1 = _round_to(_ceil_div(_ceil_div(_N1, _C), _NW), _NB1)
_WPW2 = _round_to(_ceil_div(_ceil_div(_N2, _C), _NW), _NB2)


def _window_idx(idx, n, wpw):
    tot = _NW * wpw
    out = np.empty((tot, _C), np.int32)
    for w in range(tot):
        base = min(w * _C, n - _C)
        out[w] = idx[base:base + _C]
    return out.reshape(-1)


_R1 = _window_idx(_IDX1, _N1, _WPW1)
_R2 = _window_idx(_IDX2, _N2, _WPW2)


def _sc_body(x_ref, r1_ref, r2_ref, o1_ref, o2_ref,
             idx1_v, idx2_v, buf0, buf1, buf2,
             gsem0, gsem1, gsem2, ssem0, ssem1, ssem2):
    wid = lax.axis_index("s") * _NC + lax.axis_index("c")
    bufs = (buf0, buf1, buf2)
    gsems = (gsem0, gsem1, gsem2)
    ssems = (ssem0, ssem1, ssem2)

    pltpu.sync_copy(r1_ref.at[pl.ds(wid * (_WPW1 * _C), _WPW1 * _C)], idx1_v)
    pltpu.sync_copy(r2_ref.at[pl.ds(wid * (_WPW2 * _C), _WPW2 * _C)], idx2_v)

    def phase(idx_v, out_ref, wpw, nmin, nb):
        base_w = wid * wpw

        def g_copy(r, b):
            return pltpu.make_async_copy(
                x_ref.at[idx_v.at[pl.ds(r * _C, _C)]], bufs[b], gsems[b])

        def s_copy(r, b):
            obase = jnp.minimum((base_w + r) * _C, nmin)
            return pltpu.make_async_copy(
                bufs[b], out_ref.at[pl.ds(obase, _C)], ssems[b])

        for b in range(nb):
            g_copy(b, b).start()

        def body(i, carry):
            r0 = i * nb
            for b in range(nb):
                g_copy(r0 + b, b).wait()
                s_copy(r0 + b, b).start()
            for b in range(nb):
                s_copy(r0 + b, b).wait()
                g_copy(jnp.minimum(r0 + nb + b, wpw - 1), b).start()
            return carry

        lax.fori_loop(0, wpw // nb, body, 0)
        for b in range(nb):
            g_copy(wpw - 1, b).wait()

    phase(idx1_v, o1_ref, _WPW1, _N1 - _C, _NB1)
    phase(idx2_v, o2_ref, _WPW2, _N2 - _C, _NB2)


@functools.cache
def _sc_call():
    return functools.partial(
        pl.kernel,
        out_type=(
            jax.ShapeDtypeStruct((_N1, _B, _D), jnp.float32),
            jax.ShapeDtypeStruct((_N2, _B, _D), jnp.float32),
        ),
        mesh=plsc.VectorSubcoreMesh(core_axis_name="c", subcore_axis_name="s",
                                    num_cores=_NC, num_subcores=_NS),
        scratch_types=[
            pltpu.VMEM((_WPW1 * _C,), jnp.int32),
            pltpu.VMEM((_WPW2 * _C,), jnp.int32),
            pltpu.VMEM((_C, _B, _D), jnp.float32),
            pltpu.VMEM((_C, _B, _D), jnp.float32),
            pltpu.VMEM((_C, _B, _D), jnp.float32),
            pltpu.SemaphoreType.DMA,
            pltpu.SemaphoreType.DMA,
            pltpu.SemaphoreType.DMA,
            pltpu.SemaphoreType.DMA,
            pltpu.SemaphoreType.DMA,
            pltpu.SemaphoreType.DMA,
        ],
    )(_sc_body)


def kernel(x):
    return _sc_call()(x, jnp.asarray(_R1), jnp.asarray(_R2))


# speedup vs baseline: 1.0860x; 1.0860x over previous
import functools

import jax
import jax.numpy as jnp
import numpy as np
from jax import lax
from jax.experimental import pallas as pl
from jax.experimental.pallas import tpu as pltpu
from jax.experimental.pallas import tpu_sc as plsc

_T, _B, _D = 8192, 4, 1024
_DF = _B * _D
_N1 = int(_T * 0.7)
_N2 = _T - _N1
_C = 8
_NC, _NS = 2, 16
_NW = _NC * _NS


def _ceil_div(a, b):
    return -(-a // b)


_PERM = np.asarray(jax.random.permutation(jax.random.key(42), _T))
_IDX1 = np.sort(_PERM[:_N1]).astype(np.int32)
_IDX2 = np.sort(_PERM[_N1:]).astype(np.int32)

_NB = 3


def _pick_wpw(nwin):
    wpw = _ceil_div(nwin, _NW)
    while wpw % _NB != _NB - 1:
        wpw += 1
    return wpw


_WPW1 = _pick_wpw(_ceil_div(_N1, _C))
_WPW2 = _pick_wpw(_ceil_div(_N2, _C))


def _window_idx(idx, n, wpw):
    tot = _NW * wpw
    out = np.empty((tot, _C), np.int32)
    for w in range(tot):
        base = min(w * _C, n - _C)
        out[w] = idx[base:base + _C]
    return out.reshape(-1)


_R1 = _window_idx(_IDX1, _N1, _WPW1)
_R2 = _window_idx(_IDX2, _N2, _WPW2)


def _sc_body(x_ref, r1_ref, r2_ref, o1_ref, o2_ref,
             idx1_v, idx2_v, buf0, buf1, buf2,
             gsem0, gsem1, gsem2, ssem0, ssem1, ssem2):
    wid = lax.axis_index("s") * _NC + lax.axis_index("c")
    bufs = (buf0, buf1, buf2)
    gsems = (gsem0, gsem1, gsem2)
    ssems = (ssem0, ssem1, ssem2)

    pltpu.sync_copy(r1_ref.at[pl.ds(wid * (_WPW1 * _C), _WPW1 * _C)], idx1_v)
    pltpu.sync_copy(r2_ref.at[pl.ds(wid * (_WPW2 * _C), _WPW2 * _C)], idx2_v)

    def phase(idx_v, out_ref, wpw, nmin):
        nb = _NB
        base_w = wid * wpw

        def g_copy(r, b):
            return pltpu.make_async_copy(
                x_ref.at[idx_v.at[pl.ds(r * _C, _C)]], bufs[b], gsems[b])

        def s_copy(r, b):
            obase = jnp.minimum((base_w + r) * _C, nmin)
            return pltpu.make_async_copy(
                bufs[b], out_ref.at[pl.ds(obase, _C)], ssems[b])

        for b in range(nb):
            g_copy(b, b).start()
        for r in range(nb - 1):
            g_copy(r, r % nb).wait()
            s_copy(r, r % nb).start()

        def body(i, carry):
            for j in range(nb):
                r = (nb - 1) + i * nb + j
                b = (nb - 1 + j) % nb
                g_copy(r, b).wait()
                s_copy(r, b).start()
                s_copy(r - (nb - 1), j).wait()
                g_copy(jnp.minimum(r + 1, wpw - 1), j).start()
            return carry

        lax.fori_loop(0, (wpw - (nb - 1)) // nb, body, 0)
        for r in range(wpw - (nb - 1), wpw):
            s_copy(r, r % nb).wait()
        g_copy(wpw - 1, wpw % nb).wait()

    phase(idx1_v, o1_ref, _WPW1, _N1 - _C)
    phase(idx2_v, o2_ref, _WPW2, _N2 - _C)


@functools.cache
def _sc_call():
    return functools.partial(
        pl.kernel,
        out_type=(
            jax.ShapeDtypeStruct((_N1, _B, _D), jnp.float32),
            jax.ShapeDtypeStruct((_N2, _B, _D), jnp.float32),
        ),
        mesh=plsc.VectorSubcoreMesh(core_axis_name="c", subcore_axis_name="s",
                                    num_cores=_NC, num_subcores=_NS),
        scratch_types=[
            pltpu.VMEM((_WPW1 * _C,), jnp.int32),
            pltpu.VMEM((_WPW2 * _C,), jnp.int32),
            pltpu.VMEM((_C, _B, _D), jnp.float32),
            pltpu.VMEM((_C, _B, _D), jnp.float32),
            pltpu.VMEM((_C, _B, _D), jnp.float32),
            pltpu.SemaphoreType.DMA,
            pltpu.SemaphoreType.DMA,
            pltpu.SemaphoreType.DMA,
            pltpu.SemaphoreType.DMA,
            pltpu.SemaphoreType.DMA,
            pltpu.SemaphoreType.DMA,
        ],
    )(_sc_body)


def kernel(x):
    return _sc_call()(x, jnp.asarray(_R1), jnp.asarray(_R2))


# speedup vs baseline: 1.0881x; 1.0019x over previous
import functools

import jax
import jax.numpy as jnp
import numpy as np
from jax import lax
from jax.experimental import pallas as pl
from jax.experimental.pallas import tpu as pltpu
from jax.experimental.pallas import tpu_sc as plsc

_T, _B, _D = 8192, 4, 1024
_DF = _B * _D
_N1 = int(_T * 0.7)
_N2 = _T - _N1
_C = 8
_NC, _NS = 2, 16
_NW = _NC * _NS


def _ceil_div(a, b):
    return -(-a // b)


_PERM = np.asarray(jax.random.permutation(jax.random.key(42), _T))
_IDX1 = np.sort(_PERM[:_N1]).astype(np.int32)
_IDX2 = np.sort(_PERM[_N1:]).astype(np.int32)

_NB = 3


def _pick_wpw(nwin):
    wpw = _ceil_div(nwin, _NW)
    while wpw % _NB != _NB - 1:
        wpw += 1
    return wpw


_WPW1 = _pick_wpw(_ceil_div(_N1, _C))
_WPW2 = _pick_wpw(_ceil_div(_N2, _C))


def _window_idx(idx, n, wpw):
    tot = _NW * wpw
    out = np.empty((tot, _C), np.int32)
    for w in range(tot):
        base = min(w * _C, n - _C)
        out[w] = idx[base:base + _C]
    return out.reshape(-1)


_R1 = _window_idx(_IDX1, _N1, _WPW1)
_R2 = _window_idx(_IDX2, _N2, _WPW2)


def _sc_body(x_ref, r1_ref, r2_ref, o1_ref, o2_ref,
             idx1_v, idx2_v, buf0, buf1, buf2,
             gsem0, gsem1, gsem2, ssem0, ssem1, ssem2):
    wid = lax.axis_index("s") * _NC + lax.axis_index("c")
    bufs = (buf0, buf1, buf2)
    gsems = (gsem0, gsem1, gsem2)
    ssems = (ssem0, ssem1, ssem2)

    pltpu.sync_copy(r1_ref.at[pl.ds(wid * (_WPW1 * _C), _WPW1 * _C)], idx1_v)
    pltpu.sync_copy(r2_ref.at[pl.ds(wid * (_WPW2 * _C), _WPW2 * _C)], idx2_v)

    def phase(idx_v, out_ref, wpw, nmin):
        nb = _NB
        base_w = wid * wpw

        def g_copy(r, b):
            return pltpu.make_async_copy(
                x_ref.at[idx_v.at[pl.ds(r * _C, _C)]], bufs[b], gsems[b])

        def s_copy(r, b):
            obase = jnp.minimum((base_w + r) * _C, nmin)
            return pltpu.make_async_copy(
                bufs[b], out_ref.at[pl.ds(obase, _C)], ssems[b])

        for b in range(nb):
            g_copy(b, b).start()
        for r in range(nb - 1):
            g_copy(r, r % nb).wait()
            s_copy(r, r % nb).start()

        def body(i, carry):
            for j in range(nb):
                r = (nb - 1) + i * nb + j
                b = (nb - 1 + j) % nb
                s_copy(r - (nb - 1), j).wait()
                g_copy(jnp.minimum(r + 1, wpw - 1), j).start()
                g_copy(r, b).wait()
                s_copy(r, b).start()
            return carry

        lax.fori_loop(0, (wpw - (nb - 1)) // nb, body, 0)
        for r in range(wpw - (nb - 1), wpw):
            s_copy(r, r % nb).wait()
        g_copy(wpw - 1, wpw % nb).wait()

    phase(idx1_v, o1_ref, _WPW1, _N1 - _C)
    phase(idx2_v, o2_ref, _WPW2, _N2 - _C)


@functools.cache
def _sc_call():
    return functools.partial(
        pl.kernel,
        out_type=(
            jax.ShapeDtypeStruct((_N1, _B, _D), jnp.float32),
            jax.ShapeDtypeStruct((_N2, _B, _D), jnp.float32),
        ),
        mesh=plsc.VectorSubcoreMesh(core_axis_name="c", subcore_axis_name="s",
                                    num_cores=_NC, num_subcores=_NS),
        scratch_types=[
            pltpu.VMEM((_WPW1 * _C,), jnp.int32),
            pltpu.VMEM((_WPW2 * _C,), jnp.int32),
            pltpu.VMEM((_C, _B, _D), jnp.float32),
            pltpu.VMEM((_C, _B, _D), jnp.float32),
            pltpu.VMEM((_C, _B, _D), jnp.float32),
            pltpu.SemaphoreType.DMA,
            pltpu.SemaphoreType.DMA,
            pltpu.SemaphoreType.DMA,
            pltpu.SemaphoreType.DMA,
            pltpu.SemaphoreType.DMA,
            pltpu.SemaphoreType.DMA,
        ],
    )(_sc_body)


def kernel(x):
    return _sc_call()(x, jnp.asarray(_R1), jnp.asarray(_R2))


# speedup vs baseline: 1.1287x; 1.0373x over previous
import functools

import jax
import jax.numpy as jnp
import numpy as np
from jax import lax
from jax.experimental import pallas as pl
from jax.experimental.pallas import tpu as pltpu
from jax.experimental.pallas import tpu_sc as plsc

_T, _B, _D = 8192, 4, 1024
_DF = _B * _D
_N1 = int(_T * 0.7)
_N2 = _T - _N1
_C = 12
_NC, _NS = 2, 16
_NW = _NC * _NS


def _ceil_div(a, b):
    return -(-a // b)


_PERM = np.asarray(jax.random.permutation(jax.random.key(42), _T))
_IDX1 = np.sort(_PERM[:_N1]).astype(np.int32)
_IDX2 = np.sort(_PERM[_N1:]).astype(np.int32)

_NB = 2


def _pick_wpw(nwin):
    wpw = _ceil_div(nwin, _NW)
    while wpw % _NB != _NB - 1:
        wpw += 1
    return wpw


_WPW1 = _pick_wpw(_ceil_div(_N1, _C))
_WPW2 = _pick_wpw(_ceil_div(_N2, _C))


_WST = 16


def _stride(wpw):
    return wpw * _WST


def _window_idx(idx, n, wpw):
    out = np.zeros((_NW, wpw, _WST), np.int32)
    for u in range(_NW):
        for w in range(wpw):
            base = min((u * wpw + w) * _C, n - _C)
            out[u, w, :_C] = idx[base:base + _C]
    return out.reshape(-1)


_R1 = _window_idx(_IDX1, _N1, _WPW1)
_R2 = _window_idx(_IDX2, _N2, _WPW2)


def _sc_body(x_ref, r1_ref, r2_ref, o1_ref, o2_ref,
             idx1_v, idx2_v, buf0, buf1,
             gsem0, gsem1, ssem0, ssem1):
    wid = lax.axis_index("s") * _NC + lax.axis_index("c")
    bufs = (buf0, buf1)
    gsems = (gsem0, gsem1)
    ssems = (ssem0, ssem1)

    pltpu.sync_copy(r1_ref.at[pl.ds(wid * _stride(_WPW1), _stride(_WPW1))], idx1_v)
    pltpu.sync_copy(r2_ref.at[pl.ds(wid * _stride(_WPW2), _stride(_WPW2))], idx2_v)

    def phase(idx_v, out_ref, wpw, nmin):
        nb = _NB
        base_w = wid * wpw

        def g_copy(r, b):
            return pltpu.make_async_copy(
                x_ref.at[idx_v.at[pl.ds(r * _WST, _C)]], bufs[b], gsems[b])

        def s_copy(r, b):
            obase = jnp.minimum((base_w + r) * _C, nmin)
            return pltpu.make_async_copy(
                bufs[b], out_ref.at[pl.ds(obase, _C)], ssems[b])

        for b in range(nb):
            g_copy(b, b).start()
        for r in range(nb - 1):
            g_copy(r, r % nb).wait()
            s_copy(r, r % nb).start()

        def body(i, carry):
            for j in range(nb):
                r = (nb - 1) + i * nb + j
                b = (nb - 1 + j) % nb
                g_copy(r, b).wait()
                s_copy(r, b).start()
                s_copy(r - (nb - 1), j).wait()
                g_copy(jnp.minimum(r + 1, wpw - 1), j).start()
            return carry

        lax.fori_loop(0, (wpw - (nb - 1)) // nb, body, 0)
        for r in range(wpw - (nb - 1), wpw):
            s_copy(r, r % nb).wait()
        g_copy(wpw - 1, wpw % nb).wait()

    phase(idx1_v, o1_ref, _WPW1, _N1 - _C)
    phase(idx2_v, o2_ref, _WPW2, _N2 - _C)


@functools.cache
def _sc_call():
    return functools.partial(
        pl.kernel,
        out_type=(
            jax.ShapeDtypeStruct((_N1, _B, _D), jnp.float32),
            jax.ShapeDtypeStruct((_N2, _B, _D), jnp.float32),
        ),
        mesh=plsc.VectorSubcoreMesh(core_axis_name="c", subcore_axis_name="s",
                                    num_cores=_NC, num_subcores=_NS),
        scratch_types=[
            pltpu.VMEM((_stride(_WPW1),), jnp.int32),
            pltpu.VMEM((_stride(_WPW2),), jnp.int32),
            pltpu.VMEM((_C, _B, _D), jnp.float32),
            pltpu.VMEM((_C, _B, _D), jnp.float32),
            pltpu.SemaphoreType.DMA,
            pltpu.SemaphoreType.DMA,
            pltpu.SemaphoreType.DMA,
            pltpu.SemaphoreType.DMA,
        ],
    )(_sc_body)


def kernel(x):
    return _sc_call()(x, jnp.asarray(_R1), jnp.asarray(_R2))
